# baseline (speedup 1.0000x reference)
import jax
import jax.numpy as jnp
from jax.experimental import pallas as pl
from jax.experimental.pallas import tpu as pltpu

_BLOCK = 512


def _router_kernel(x_ref, w_ref, o_ref):
    logits = jax.lax.dot_general(
        w_ref[...],
        x_ref[...],
        dimension_numbers=(((1,), (1,)), ((), ())),
        preferred_element_type=jnp.float32,
    )
    m = jnp.max(logits, axis=0, keepdims=True)
    e = jnp.exp(logits - m)
    o_ref[...] = e / jnp.sum(e, axis=0, keepdims=True)


def kernel(x, W):
    n_tokens, in_dim = x.shape
    n_experts = W.shape[0]
    out_t = pl.pallas_call(
        _router_kernel,
        grid=(n_tokens // _BLOCK,),
        in_specs=[
            pl.BlockSpec((_BLOCK, in_dim), lambda i: (i, 0)),
            pl.BlockSpec((n_experts, in_dim), lambda i: (0, 0)),
        ],
        out_specs=pl.BlockSpec((n_experts, _BLOCK), lambda i: (0, i)),
        out_shape=jax.ShapeDtypeStruct((n_experts, n_tokens), jnp.float32),
        compiler_params=pltpu.CompilerParams(
            dimension_semantics=("arbitrary",)
        ),
    )(x, W)
    return out_t.T


# speedup vs baseline: 1.2010x; 1.2010x over previous
import jax
import jax.numpy as jnp
from jax.experimental import pallas as pl
from jax.experimental.pallas import tpu as pltpu

_BLOCK = 1024


def _router_kernel(x_ref, w_ref, o_ref):
    logits = jax.lax.dot_general(
        w_ref[...],
        x_ref[...],
        dimension_numbers=(((1,), (1,)), ((), ())),
        preferred_element_type=jnp.float32,
    )
    m = jnp.max(logits, axis=0, keepdims=True)
    e = jnp.exp(logits - m)
    o_ref[...] = e / jnp.sum(e, axis=0, keepdims=True)


def kernel(x, W):
    n_tokens, in_dim = x.shape
    n_experts = W.shape[0]
    out_t = pl.pallas_call(
        _router_kernel,
        grid=(n_tokens // _BLOCK,),
        in_specs=[
            pl.BlockSpec((_BLOCK, in_dim), lambda i: (i, 0)),
            pl.BlockSpec((n_experts, in_dim), lambda i: (0, 0)),
        ],
        out_specs=pl.BlockSpec((n_experts, _BLOCK), lambda i: (0, i)),
        out_shape=jax.ShapeDtypeStruct((n_experts, n_tokens), jnp.float32),
        compiler_params=pltpu.CompilerParams(
            dimension_semantics=("arbitrary",)
        ),
    )(x, W)
    return out_t.T
